# Initial kernel scaffold; baseline (speedup 1.0000x reference)
import functools

import jax
import jax.numpy as jnp
from jax import lax
from jax.experimental import pallas as pl
from jax.experimental.pallas import tpu as pltpu
from jax.experimental.pallas import tpu_sc as plsc

N = 10000
E = 320000
D = 128
DOUT = 16

NW = 32
R = 320
NPAD = NW * R
FLUSH = 1024
STAGE = FLUSH + 16
EPAD = (E // FLUSH + 2) * FLUSH
BLK = 16000
G = 128

_MESH = plsc.VectorSubcoreMesh(core_axis_name="c", subcore_axis_name="s")


def _wid():
    return lax.axis_index("s") * 2 + lax.axis_index("c")


def _scan_body(src_hbm, dst_hbm, list_hbm, cnt_hbm, svb, dvb, stage, cntv, sem):
    del sem
    wid = _wid()
    lo = wid * R

    def block(b, carry):
        pltpu.sync_copy(src_hbm.at[pl.ds(b * BLK, BLK)], svb)
        pltpu.sync_copy(dst_hbm.at[pl.ds(b * BLK, BLK)], dvb)

        def group(j, carry):
            f, nf = carry
            dv = dvb[pl.ds(j * 16, 16)]
            sv = svb[pl.ds(j * 16, 16)]
            dl = dv - lo
            mask = (dl >= 0) & (dl < R)
            packed = (sv << 9) | (dl & 511)
            plsc.store_compressed(stage.at[pl.ds(f, 16)], packed, mask=mask)
            f = f + jnp.max(plsc.all_reduce_population_count(mask))
            do = f >= FLUSH

            @pl.when(do)
            def _():
                pltpu.sync_copy(stage.at[pl.ds(0, FLUSH)],
                                list_hbm.at[wid, pl.ds(nf * FLUSH, FLUSH)])
                stage[pl.ds(0, 16)] = stage[pl.ds(FLUSH, 16)]

            f = jnp.where(do, f - FLUSH, f)
            nf = nf + do.astype(jnp.int32)
            return f, nf

        return lax.fori_loop(0, BLK // 16, group, carry)

    f, nf = lax.fori_loop(0, E // BLK, block, (jnp.int32(0), jnp.int32(0)))
    pltpu.sync_copy(stage.at[pl.ds(0, FLUSH)],
                    list_hbm.at[wid, pl.ds(nf * FLUSH, FLUSH)])
    cntv[...] = jnp.full((16,), nf * FLUSH + f, jnp.int32)
    pltpu.sync_copy(cntv, cnt_hbm.at[wid])


@jax.jit
def _scan(src, dst):
    kern = pl.kernel(
        _scan_body,
        out_type=(jax.ShapeDtypeStruct((NW, EPAD), jnp.int32),
                  jax.ShapeDtypeStruct((NW, 16), jnp.int32)),
        mesh=_MESH,
        scratch_types=[
            pltpu.VMEM((BLK,), jnp.int32),
            pltpu.VMEM((BLK,), jnp.int32),
            pltpu.VMEM((STAGE,), jnp.int32),
            pltpu.VMEM((16,), jnp.int32),
            pltpu.SemaphoreType.DMA,
        ],
    )
    return kern(src, dst)


def _segmax_body(list_hbm, cnt_hbm, hp_hbm, agg_hbm,
                 table, rows, lch, idxv, dstlv, cntv, sem):
    wid = _wid()
    zeros16 = jnp.zeros((16,), jnp.float32)
    iota16 = lax.iota(jnp.int32, 16)

    @pl.loop(0, R)
    def _(r):
        for k in range(8):
            table[r, pl.ds(k * 16, 16)] = zeros16

    pltpu.sync_copy(cnt_hbm.at[wid], cntv)
    cnt = jnp.max(cntv)
    nch = (cnt + G - 1) // G

    def chunk(ch, _):
        base = ch * G
        pltpu.sync_copy(list_hbm.at[wid, pl.ds(base, G)], lch)

        def unp(j, _):
            v = lch[pl.ds(j * 16, 16)]
            iv = jnp.minimum(lax.shift_right_logical(v, 9), N - 1)
            idxv[pl.ds(j * 16, 16)] = iv
            dstlv[pl.ds(j * 16, 16)] = v & 511
            return 0

        lax.fori_loop(0, G // 16, unp, 0)
        pltpu.async_copy(hp_hbm.at[idxv], rows, sem).wait()
        rem = jnp.minimum(cnt - base, G)

        def grp(jg, _):
            dv16 = dstlv[pl.ds(jg * 16, 16)]
            glen = jnp.minimum(rem - jg * 16, 16)

            def edge(l, _):
                d = jnp.max(jnp.where(iota16 == l, dv16, -1))
                e = jg * 16 + l
                for k in range(8):
                    sl = pl.ds(k * 16, 16)
                    table[d, sl] = jnp.maximum(table[d, sl], rows[e, sl])
                return 0

            return lax.fori_loop(0, glen, edge, 0)

        lax.fori_loop(0, (rem + 15) // 16, grp, 0)
        return 0

    lax.fori_loop(0, nch, chunk, 0)
    pltpu.sync_copy(table, agg_hbm.at[pl.ds(wid * R, R)])


@jax.jit
def _segmax(lists, counts, hp):
    kern = pl.kernel(
        _segmax_body,
        out_type=jax.ShapeDtypeStruct((NPAD, D), jnp.float32),
        mesh=_MESH,
        scratch_types=[
            pltpu.VMEM((R, D), jnp.float32),
            pltpu.VMEM((G, D), jnp.float32),
            pltpu.VMEM((G,), jnp.int32),
            pltpu.VMEM((G,), jnp.int32),
            pltpu.VMEM((G,), jnp.int32),
            pltpu.VMEM((16,), jnp.int32),
            pltpu.SemaphoreType.DMA,
        ],
    )
    return kern(lists, counts, hp)


def _mm(a, w):
    return lax.dot_general(a, w, (((1,), (1,)), ((), ())),
                           preferred_element_type=jnp.float32,
                           precision=lax.Precision.HIGHEST)


def _dense1_body(x_ref, wp_ref, bp_ref, hp_ref):
    hp_ref[...] = jnp.maximum(_mm(x_ref[...], wp_ref[...]) + bp_ref[...], 0.0)


def _dense2_body(x_ref, a1_ref, ws_ref, wn_ref, b_ref, wp2_ref, bp2_ref,
                 h1_ref, hp2_ref):
    h = _mm(x_ref[...], ws_ref[...]) + _mm(a1_ref[...], wn_ref[...]) + b_ref[...]
    h = jnp.where(h >= 0.0, h, 0.01 * h)
    h1_ref[...] = h
    hp2_ref[...] = jnp.maximum(_mm(h, wp2_ref[...]) + bp2_ref[...], 0.0)


def _dense3_body(h1_ref, a2_ref, ws_ref, wn_ref, b_ref, o_ref):
    o_ref[...] = (_mm(h1_ref[...], ws_ref[...]) + _mm(a2_ref[...], wn_ref[...])
                  + b_ref[...])


@jax.jit
def _dense1(x, wp, bp):
    return pl.pallas_call(
        _dense1_body,
        out_shape=jax.ShapeDtypeStruct((N, D), jnp.float32),
    )(x, wp, bp)


@jax.jit
def _dense2(x, a1, ws, wn, b, wp2, bp2):
    return pl.pallas_call(
        _dense2_body,
        out_shape=(jax.ShapeDtypeStruct((N, D), jnp.float32),
                   jax.ShapeDtypeStruct((N, D), jnp.float32)),
    )(x, a1, ws, wn, b, wp2, bp2)


@jax.jit
def _dense3(h1, a2, ws, wn, b):
    return pl.pallas_call(
        _dense3_body,
        out_shape=jax.ShapeDtypeStruct((N, DOUT), jnp.float32),
    )(h1, a2, ws, wn, b)


def kernel(x, edge_index, W_pool1, b_pool1, W_self1, W_neigh1, b1,
           W_pool2, b_pool2, W_self2, W_neigh2, b2):
    src = edge_index[0]
    dst = edge_index[1]
    lists, counts = _scan(src, dst)
    hp1 = _dense1(x, W_pool1, b_pool1.reshape(1, D))
    agg1 = _segmax(lists, counts, hp1)[:N]
    h1, hp2 = _dense2(x, agg1, W_self1, W_neigh1, b1.reshape(1, D),
                      W_pool2, b_pool2.reshape(1, D))
    agg2 = _segmax(lists, counts, hp2)[:N]
    return _dense3(h1, agg2, W_self2, W_neigh2, b2.reshape(1, DOUT))


# speedup vs baseline: 2.1894x; 2.1894x over previous
import dataclasses
import functools

import jax
import jax.numpy as jnp
from jax import lax
from jax.experimental import pallas as pl
from jax.experimental.pallas import tpu as pltpu
from jax.experimental.pallas import tpu_sc as plsc

N = 10000
E = 320000
D = 128
DOUT = 16

NW = 32
R = 320
NPAD = NW * R
FLUSH = 1024
STAGE = FLUSH + 16
EPAD = (E // FLUSH + 2) * FLUSH
BLK = 16000
G = 128

_MESH = plsc.VectorSubcoreMesh(core_axis_name="c", subcore_axis_name="s")

_SC_PARAMS = pltpu.CompilerParams()
if "needs_layout_passes" in pltpu.CompilerParams.__dataclass_fields__:
    _SC_PARAMS = dataclasses.replace(_SC_PARAMS, needs_layout_passes=False)


def _wid():
    return lax.axis_index("s") * 2 + lax.axis_index("c")


def _scan_body(src_hbm, dst_hbm, list_hbm, cnt_hbm, svb, dvb, stage, cntv, sem):
    del sem
    wid = _wid()
    lo = wid * R

    def block(b, carry):
        pltpu.sync_copy(src_hbm.at[pl.ds(b * BLK, BLK)], svb)
        pltpu.sync_copy(dst_hbm.at[pl.ds(b * BLK, BLK)], dvb)

        def group(j, carry):
            f, nf = carry
            dv = dvb[pl.ds(j * 16, 16)]
            sv = svb[pl.ds(j * 16, 16)]
            dl = dv - lo
            mask = (dl >= 0) & (dl < R)
            packed = (sv << 9) | (dl & 511)
            plsc.store_compressed(stage.at[pl.ds(f, 16)], packed, mask=mask)
            f = f + jnp.max(plsc.all_reduce_population_count(mask))
            do = f >= FLUSH

            @pl.when(do)
            def _():
                pltpu.sync_copy(stage.at[pl.ds(0, FLUSH)],
                                list_hbm.at[wid, pl.ds(nf * FLUSH, FLUSH)])
                stage[pl.ds(0, 16)] = stage[pl.ds(FLUSH, 16)]

            f = jnp.where(do, f - FLUSH, f)
            nf = nf + do.astype(jnp.int32)
            return f, nf

        return lax.fori_loop(0, BLK // 16, group, carry)

    f, nf = lax.fori_loop(0, E // BLK, block, (jnp.int32(0), jnp.int32(0)))
    pltpu.sync_copy(stage.at[pl.ds(0, FLUSH)],
                    list_hbm.at[wid, pl.ds(nf * FLUSH, FLUSH)])
    cntv[...] = jnp.full((16,), nf * FLUSH + f, jnp.int32)
    pltpu.sync_copy(cntv, cnt_hbm.at[wid])


@jax.jit
def _scan(src, dst):
    kern = pl.kernel(
        _scan_body,
        out_type=(jax.ShapeDtypeStruct((NW, EPAD), jnp.int32),
                  jax.ShapeDtypeStruct((NW, 16), jnp.int32)),
        mesh=_MESH,
        scratch_types=[
            pltpu.VMEM((BLK,), jnp.int32),
            pltpu.VMEM((BLK,), jnp.int32),
            pltpu.VMEM((STAGE,), jnp.int32),
            pltpu.VMEM((16,), jnp.int32),
            pltpu.SemaphoreType.DMA,
        ],
        compiler_params=_SC_PARAMS,
    )
    return kern(src, dst)


def _segmax_body(list_hbm, cnt_hbm, hp_hbm, agg_hbm,
                 table, rows, lch, idxv, dstlv, cntv, sem):
    wid = _wid()
    zeros16 = jnp.zeros((16,), jnp.float32)
    iota16 = lax.iota(jnp.int32, 16)

    @pl.loop(0, R)
    def _(r):
        for k in range(8):
            table[r, pl.ds(k * 16, 16)] = zeros16

    pltpu.sync_copy(cnt_hbm.at[wid], cntv)
    cnt = jnp.max(cntv[...])
    nch = (cnt + G - 1) // G

    def chunk(ch, _):
        base = ch * G
        pltpu.sync_copy(list_hbm.at[wid, pl.ds(base, G)], lch)

        def unp(j, _):
            v = lch[pl.ds(j * 16, 16)]
            iv = jnp.minimum(lax.shift_right_logical(v, 9), N - 1)
            idxv[pl.ds(j * 16, 16)] = iv
            dstlv[pl.ds(j * 16, 16)] = v & 511
            return 0

        lax.fori_loop(0, G // 16, unp, 0)
        pltpu.async_copy(hp_hbm.at[idxv], rows, sem).wait()
        rem = jnp.minimum(cnt - base, G)

        def grp(jg, _):
            dv16 = dstlv[pl.ds(jg * 16, 16)]
            glen = jnp.minimum(rem - jg * 16, 16)

            def edge(l, _):
                d = jnp.max(jnp.where(iota16 == l, dv16, -1))
                e = jg * 16 + l
                for k in range(8):
                    sl = pl.ds(k * 16, 16)
                    table[d, sl] = jnp.maximum(table[d, sl], rows[e, sl])
                return 0

            return lax.fori_loop(0, glen, edge, 0)

        lax.fori_loop(0, (rem + 15) // 16, grp, 0)
        return 0

    lax.fori_loop(0, nch, chunk, 0)
    pltpu.sync_copy(table, agg_hbm.at[pl.ds(wid * R, R)])


@jax.jit
def _segmax(lists, counts, hp):
    kern = pl.kernel(
        _segmax_body,
        out_type=jax.ShapeDtypeStruct((NPAD, D), jnp.float32),
        mesh=_MESH,
        scratch_types=[
            pltpu.VMEM((R, D), jnp.float32),
            pltpu.VMEM((G, D), jnp.float32),
            pltpu.VMEM((G,), jnp.int32),
            pltpu.VMEM((G,), jnp.int32),
            pltpu.VMEM((G,), jnp.int32),
            pltpu.VMEM((16,), jnp.int32),
            pltpu.SemaphoreType.DMA,
        ],
        compiler_params=_SC_PARAMS,
    )
    return kern(lists, counts, hp)


def _mm(a, w):
    return lax.dot_general(a, w, (((1,), (1,)), ((), ())),
                           preferred_element_type=jnp.float32,
                           precision=lax.Precision.HIGHEST)


def _dense1_body(x_ref, wp_ref, bp_ref, hp_ref):
    hp_ref[...] = jnp.maximum(_mm(x_ref[...], wp_ref[...]) + bp_ref[...], 0.0)


def _dense2_body(x_ref, a1_ref, ws_ref, wn_ref, b_ref, wp2_ref, bp2_ref,
                 h1_ref, hp2_ref):
    h = _mm(x_ref[...], ws_ref[...]) + _mm(a1_ref[...], wn_ref[...]) + b_ref[...]
    h = jnp.where(h >= 0.0, h, 0.01 * h)
    h1_ref[...] = h
    hp2_ref[...] = jnp.maximum(_mm(h, wp2_ref[...]) + bp2_ref[...], 0.0)


def _dense3_body(h1_ref, a2_ref, ws_ref, wn_ref, b_ref, o_ref):
    o_ref[...] = (_mm(h1_ref[...], ws_ref[...]) + _mm(a2_ref[...], wn_ref[...])
                  + b_ref[...])


@jax.jit
def _dense1(x, wp, bp):
    return pl.pallas_call(
        _dense1_body,
        out_shape=jax.ShapeDtypeStruct((N, D), jnp.float32),
    )(x, wp, bp)


@jax.jit
def _dense2(x, a1, ws, wn, b, wp2, bp2):
    return pl.pallas_call(
        _dense2_body,
        out_shape=(jax.ShapeDtypeStruct((N, D), jnp.float32),
                   jax.ShapeDtypeStruct((N, D), jnp.float32)),
    )(x, a1, ws, wn, b, wp2, bp2)


@jax.jit
def _dense3(h1, a2, ws, wn, b):
    return pl.pallas_call(
        _dense3_body,
        out_shape=jax.ShapeDtypeStruct((N, DOUT), jnp.float32),
    )(h1, a2, ws, wn, b)


def kernel(x, edge_index, W_pool1, b_pool1, W_self1, W_neigh1, b1,
           W_pool2, b_pool2, W_self2, W_neigh2, b2):
    src = edge_index[0]
    dst = edge_index[1]
    lists, counts = _scan(src, dst)
    hp1 = _dense1(x, W_pool1, b_pool1.reshape(1, D))
    agg1 = _segmax(lists, counts, hp1)[:N]
    h1, hp2 = _dense2(x, agg1, W_self1, W_neigh1, b1.reshape(1, D),
                      W_pool2, b_pool2.reshape(1, D))
    agg2 = _segmax(lists, counts, hp2)[:N]
    return _dense3(h1, agg2, W_self2, W_neigh2, b2.reshape(1, DOUT))


# speedup vs baseline: 2.4778x; 1.1317x over previous
import dataclasses
import functools

import jax
import jax.numpy as jnp
from jax import lax
from jax.experimental import pallas as pl
from jax.experimental.pallas import tpu as pltpu
from jax.experimental.pallas import tpu_sc as plsc

N = 10000
E = 320000
D = 128
DOUT = 16

NW = 32
R = 320
NPAD = NW * R
FLUSH = 1024
STAGE = FLUSH + 16
EPAD = (E // FLUSH + 2) * FLUSH
BLK = 16000
G = 256

_MESH = plsc.VectorSubcoreMesh(core_axis_name="c", subcore_axis_name="s")

_SC_PARAMS = pltpu.CompilerParams()
if "needs_layout_passes" in pltpu.CompilerParams.__dataclass_fields__:
    _SC_PARAMS = dataclasses.replace(_SC_PARAMS, needs_layout_passes=False)


def _wid():
    return lax.axis_index("s") * 2 + lax.axis_index("c")


DUMMY = 511


def _scan_body(src_hbm, dst_hbm, list_hbm, cnt_hbm,
               svb0, dvb0, svb1, dvb1, stage, cntv, sem0, sem1):
    wid = _wid()
    lo = wid * R
    nblk = E // BLK
    bufs = ((svb0, dvb0, sem0), (svb1, dvb1, sem1))

    def start_load(b, sv_b, dv_b, sem_b):
        pltpu.make_async_copy(src_hbm.at[pl.ds(b * BLK, BLK)], sv_b, sem_b).start()
        pltpu.make_async_copy(dst_hbm.at[pl.ds(b * BLK, BLK)], dv_b, sem_b).start()

    def wait_load(b, sv_b, dv_b, sem_b):
        pltpu.make_async_copy(src_hbm.at[pl.ds(b * BLK, BLK)], sv_b, sem_b).wait()
        pltpu.make_async_copy(dst_hbm.at[pl.ds(b * BLK, BLK)], dv_b, sem_b).wait()

    start_load(0, *bufs[0])

    def process_block(b, sv_b, dv_b, carry):
        def group4(jj, carry):
            for u in range(4):
                f, nf = carry
                j = jj * 4 + u
                dv = dv_b[pl.ds(j * 16, 16)]
                sv = sv_b[pl.ds(j * 16, 16)]
                dl = dv - lo
                mask = dl.astype(jnp.uint32) < jnp.uint32(R)
                packed = (sv << 9) | dl
                plsc.store_compressed(stage.at[pl.ds(f, 16)], packed, mask=mask)
                f = f + plsc.all_reduce_population_count(mask)[0]
                do = f >= FLUSH

                @pl.when(do)
                def _():
                    pltpu.sync_copy(stage.at[pl.ds(0, FLUSH)],
                                    list_hbm.at[wid, pl.ds(nf * FLUSH, FLUSH)])
                    stage[pl.ds(0, 16)] = stage[pl.ds(FLUSH, 16)]

                f = jnp.where(do, f - FLUSH, f)
                nf = nf + do.astype(jnp.int32)
                carry = (f, nf)
            return carry

        return lax.fori_loop(0, BLK // 64, group4, carry)

    def block_pair(bb, carry):
        for p in range(2):
            b = bb * 2 + p

            @pl.when(b < nblk)
            def _():
                wait_load(b, *bufs[p])

                @pl.when(b + 1 < nblk)
                def _():
                    start_load(b + 1, *bufs[1 - p])

            carry = lax.cond(b < nblk,
                             lambda c: process_block(b, bufs[p][0], bufs[p][1], c),
                             lambda c: c, carry)
        return carry

    f, nf = lax.fori_loop(0, (nblk + 1) // 2, block_pair,
                          (jnp.int32(0), jnp.int32(0)))

    iota16 = lax.iota(jnp.int32, 16)

    @pl.loop(0, STAGE // 16)
    def _(j):
        g = j * 16
        sl = pl.ds(g, 16)
        stage[sl] = jnp.where(g + iota16 >= f, DUMMY, stage[sl])

    pltpu.sync_copy(stage.at[pl.ds(0, FLUSH)],
                    list_hbm.at[wid, pl.ds(nf * FLUSH, FLUSH)])
    cntv[...] = jnp.full((16,), nf * FLUSH + f, jnp.int32)
    pltpu.sync_copy(cntv, cnt_hbm.at[wid])


@jax.jit
def _scan(src, dst):
    kern = pl.kernel(
        _scan_body,
        out_type=(jax.ShapeDtypeStruct((NW, EPAD), jnp.int32),
                  jax.ShapeDtypeStruct((NW, 16), jnp.int32)),
        mesh=_MESH,
        scratch_types=[
            pltpu.VMEM((BLK,), jnp.int32),
            pltpu.VMEM((BLK,), jnp.int32),
            pltpu.VMEM((BLK,), jnp.int32),
            pltpu.VMEM((BLK,), jnp.int32),
            pltpu.VMEM((STAGE,), jnp.int32),
            pltpu.VMEM((16,), jnp.int32),
            pltpu.SemaphoreType.DMA,
            pltpu.SemaphoreType.DMA,
        ],
        compiler_params=_SC_PARAMS,
    )
    return kern(src, dst)


def _segmax_body(list_hbm, cnt_hbm, hp_hbm, agg_hbm,
                 table, rows0, rows1, idx0, idx1, dst0, dst1, lch, cntv,
                 sem0, sem1):
    wid = _wid()
    zeros16 = jnp.zeros((16,), jnp.float32)

    @pl.loop(0, R + 1)
    def _(r):
        for k in range(8):
            table[r, pl.ds(k * 16, 16)] = zeros16

    pltpu.sync_copy(cnt_hbm.at[wid], cntv)
    cnt = jnp.max(cntv[...])
    nch = (cnt + G - 1) // G
    bufs = ((rows0, idx0, dst0, sem0), (rows1, idx1, dst1, sem1))

    def load_unpack_start(ch, rows_b, idx_b, dst_b, sem_b):
        pltpu.sync_copy(list_hbm.at[wid, pl.ds(ch * G, G)], lch)
        for j in range(G // 16):
            sl = pl.ds(j * 16, 16)
            v = lch[sl]
            idx_b[sl] = jnp.minimum(lax.shift_right_logical(v, 9), N - 1)
            dst_b[sl] = jnp.minimum(v & 511, R)
        pltpu.make_async_copy(hp_hbm.at[idx_b], rows_b, sem_b).start()

    def rmw(rows_b, idx_b, dst_b, sem_b):
        pltpu.make_async_copy(hp_hbm.at[idx_b], rows_b, sem_b).wait()

        @pl.loop(0, G // 16)
        def _(jg):
            dv16 = dst_b[pl.ds(jg * 16, 16)]
            for l in range(16):
                d = dv16[l]
                e = jg * 16 + l
                for k in range(8):
                    sl = pl.ds(k * 16, 16)
                    table[d, sl] = jnp.maximum(table[d, sl], rows_b[e, sl])

    @pl.when(nch > 0)
    def _():
        load_unpack_start(0, *bufs[0])

    def chunk_pair(it, _):
        for p in range(2):
            ch = it * 2 + p

            @pl.when(ch < nch)
            def _():
                @pl.when(ch + 1 < nch)
                def _():
                    load_unpack_start(ch + 1, *bufs[1 - p])

                rmw(*bufs[p])
        return 0

    lax.fori_loop(0, (nch + 1) // 2, chunk_pair, 0)
    pltpu.sync_copy(table.at[pl.ds(0, R)], agg_hbm.at[pl.ds(wid * R, R)])


@jax.jit
def _segmax(lists, counts, hp):
    kern = pl.kernel(
        _segmax_body,
        out_type=jax.ShapeDtypeStruct((NPAD, D), jnp.float32),
        mesh=_MESH,
        scratch_types=[
            pltpu.VMEM((R + 1, D), jnp.float32),
            pltpu.VMEM((G, D), jnp.float32),
            pltpu.VMEM((G, D), jnp.float32),
            pltpu.VMEM((G,), jnp.int32),
            pltpu.VMEM((G,), jnp.int32),
            pltpu.VMEM((G,), jnp.int32),
            pltpu.VMEM((G,), jnp.int32),
            pltpu.VMEM((G,), jnp.int32),
            pltpu.VMEM((16,), jnp.int32),
            pltpu.SemaphoreType.DMA,
            pltpu.SemaphoreType.DMA,
        ],
        compiler_params=_SC_PARAMS,
    )
    return kern(lists, counts, hp)


def _mm(a, w):
    return lax.dot_general(a, w, (((1,), (1,)), ((), ())),
                           preferred_element_type=jnp.float32,
                           precision=lax.Precision.HIGHEST)


def _dense1_body(x_ref, wp_ref, bp_ref, hp_ref):
    hp_ref[...] = jnp.maximum(_mm(x_ref[...], wp_ref[...]) + bp_ref[...], 0.0)


def _dense2_body(x_ref, a1_ref, ws_ref, wn_ref, b_ref, wp2_ref, bp2_ref,
                 h1_ref, hp2_ref):
    h = _mm(x_ref[...], ws_ref[...]) + _mm(a1_ref[...], wn_ref[...]) + b_ref[...]
    h = jnp.where(h >= 0.0, h, 0.01 * h)
    h1_ref[...] = h
    hp2_ref[...] = jnp.maximum(_mm(h, wp2_ref[...]) + bp2_ref[...], 0.0)


def _dense3_body(h1_ref, a2_ref, ws_ref, wn_ref, b_ref, o_ref):
    o_ref[...] = (_mm(h1_ref[...], ws_ref[...]) + _mm(a2_ref[...], wn_ref[...])
                  + b_ref[...])


@jax.jit
def _dense1(x, wp, bp):
    return pl.pallas_call(
        _dense1_body,
        out_shape=jax.ShapeDtypeStruct((N, D), jnp.float32),
    )(x, wp, bp)


@jax.jit
def _dense2(x, a1, ws, wn, b, wp2, bp2):
    return pl.pallas_call(
        _dense2_body,
        out_shape=(jax.ShapeDtypeStruct((N, D), jnp.float32),
                   jax.ShapeDtypeStruct((N, D), jnp.float32)),
    )(x, a1, ws, wn, b, wp2, bp2)


@jax.jit
def _dense3(h1, a2, ws, wn, b):
    return pl.pallas_call(
        _dense3_body,
        out_shape=jax.ShapeDtypeStruct((N, DOUT), jnp.float32),
    )(h1, a2, ws, wn, b)


def kernel(x, edge_index, W_pool1, b_pool1, W_self1, W_neigh1, b1,
           W_pool2, b_pool2, W_self2, W_neigh2, b2):
    src = edge_index[0]
    dst = edge_index[1]
    lists, counts = _scan(src, dst)
    hp1 = _dense1(x, W_pool1, b_pool1.reshape(1, D))
    agg1 = _segmax(lists, counts, hp1)[:N]
    h1, hp2 = _dense2(x, agg1, W_self1, W_neigh1, b1.reshape(1, D),
                      W_pool2, b_pool2.reshape(1, D))
    agg2 = _segmax(lists, counts, hp2)[:N]
    return _dense3(h1, agg2, W_self2, W_neigh2, b2.reshape(1, DOUT))


# speedup vs baseline: 4.7545x; 1.9189x over previous
import dataclasses
import functools

import jax
import jax.numpy as jnp
from jax import lax
from jax.experimental import pallas as pl
from jax.experimental.pallas import tpu as pltpu
from jax.experimental.pallas import tpu_sc as plsc

N = 10000
E = 320000
D = 128
DOUT = 16

NW = 32
R = 320
NPAD = NW * R
FLUSH = 1024
STAGE = FLUSH + 96
EPAD = (E // FLUSH + 2) * FLUSH
BLK = 16000
G = 256

_MESH = plsc.VectorSubcoreMesh(core_axis_name="c", subcore_axis_name="s")

_SC_PARAMS = pltpu.CompilerParams()
if "needs_layout_passes" in pltpu.CompilerParams.__dataclass_fields__:
    _SC_PARAMS = dataclasses.replace(_SC_PARAMS, needs_layout_passes=False)


def _wid():
    return lax.axis_index("s") * 2 + lax.axis_index("c")


DUMMY = 511


def _scan_body(src_hbm, dst_hbm, list_hbm, cnt_hbm,
               svb0, dvb0, svb1, dvb1, stage, cntv, sem0, sem1):
    wid = _wid()
    lo = wid * R
    nblk = E // BLK
    bufs = ((svb0, dvb0, sem0), (svb1, dvb1, sem1))

    def start_load(b, sv_b, dv_b, sem_b):
        pltpu.make_async_copy(src_hbm.at[pl.ds(b * BLK, BLK)], sv_b, sem_b).start()
        pltpu.make_async_copy(dst_hbm.at[pl.ds(b * BLK, BLK)], dv_b, sem_b).start()

    def wait_load(b, sv_b, dv_b, sem_b):
        pltpu.make_async_copy(src_hbm.at[pl.ds(b * BLK, BLK)], sv_b, sem_b).wait()
        pltpu.make_async_copy(dst_hbm.at[pl.ds(b * BLK, BLK)], dv_b, sem_b).wait()

    start_load(0, *bufs[0])

    def process_block(b, sv_b, dv_b, carry):
        def group4(jj, carry):
            f, nf = carry
            masks, packeds, pcs = [], [], []
            for u in range(4):
                j = jj * 4 + u
                dv = dv_b[pl.ds(j * 16, 16)]
                sv = sv_b[pl.ds(j * 16, 16)]
                dl = dv - lo
                m = dl.astype(jnp.uint32) < jnp.uint32(R)
                masks.append(m)
                packeds.append((sv << 9) | dl)
                pcs.append(plsc.all_reduce_population_count(m))
            for u in range(4):
                plsc.store_compressed(stage.at[pl.ds(f, 16)], packeds[u],
                                      mask=masks[u])
                f = f + pcs[u][0]
            do = f >= FLUSH

            @pl.when(do)
            def _():
                pltpu.sync_copy(stage.at[pl.ds(0, FLUSH)],
                                list_hbm.at[wid, pl.ds(nf * FLUSH, FLUSH)])
                for t in range(5):
                    stage[pl.ds(t * 16, 16)] = stage[pl.ds(FLUSH + t * 16, 16)]

            f = jnp.where(do, f - FLUSH, f)
            nf = nf + do.astype(jnp.int32)
            return f, nf

        return lax.fori_loop(0, BLK // 64, group4, carry)

    def block_pair(bb, carry):
        for p in range(2):
            b = bb * 2 + p

            @pl.when(b < nblk)
            def _():
                wait_load(b, *bufs[p])

                @pl.when(b + 1 < nblk)
                def _():
                    start_load(b + 1, *bufs[1 - p])

            carry = lax.cond(b < nblk,
                             lambda c: process_block(b, bufs[p][0], bufs[p][1], c),
                             lambda c: c, carry)
        return carry

    f, nf = lax.fori_loop(0, (nblk + 1) // 2, block_pair,
                          (jnp.int32(0), jnp.int32(0)))

    iota16 = lax.iota(jnp.int32, 16)

    @pl.loop(0, STAGE // 16)
    def _(j):
        g = j * 16
        sl = pl.ds(g, 16)
        stage[sl] = jnp.where(g + iota16 >= f, DUMMY, stage[sl])

    pltpu.sync_copy(stage.at[pl.ds(0, FLUSH)],
                    list_hbm.at[wid, pl.ds(nf * FLUSH, FLUSH)])
    cntv[...] = jnp.full((16,), nf * FLUSH + f, jnp.int32)
    pltpu.sync_copy(cntv, cnt_hbm.at[wid])


@jax.jit
def _scan(src, dst):
    kern = pl.kernel(
        _scan_body,
        out_type=(jax.ShapeDtypeStruct((NW, EPAD), jnp.int32),
                  jax.ShapeDtypeStruct((NW, 16), jnp.int32)),
        mesh=_MESH,
        scratch_types=[
            pltpu.VMEM((BLK,), jnp.int32),
            pltpu.VMEM((BLK,), jnp.int32),
            pltpu.VMEM((BLK,), jnp.int32),
            pltpu.VMEM((BLK,), jnp.int32),
            pltpu.VMEM((STAGE,), jnp.int32),
            pltpu.VMEM((16,), jnp.int32),
            pltpu.SemaphoreType.DMA,
            pltpu.SemaphoreType.DMA,
        ],
        compiler_params=_SC_PARAMS,
    )
    return kern(src, dst)


def _segmax_body(list_hbm, cnt_hbm, hp_hbm, agg_hbm,
                 table, rows0, rows1, idx0, idx1, dst0, dst1, lch, cntv,
                 sem0, sem1):
    wid = _wid()
    zeros16 = jnp.zeros((16,), jnp.float32)

    @pl.loop(0, R + 1)
    def _(r):
        for k in range(8):
            table[r, pl.ds(k * 16, 16)] = zeros16

    pltpu.sync_copy(cnt_hbm.at[wid], cntv)
    cnt = jnp.max(cntv[...])
    nch = (cnt + G - 1) // G
    bufs = ((rows0, idx0, dst0, sem0), (rows1, idx1, dst1, sem1))

    def load_unpack_start(ch, rows_b, idx_b, dst_b, sem_b):
        pltpu.sync_copy(list_hbm.at[wid, pl.ds(ch * G, G)], lch)
        for j in range(G // 16):
            sl = pl.ds(j * 16, 16)
            v = lch[sl]
            idx_b[sl] = jnp.minimum(lax.shift_right_logical(v, 9), N - 1)
            dst_b[sl] = jnp.minimum(v & 511, R)
        pltpu.make_async_copy(hp_hbm.at[idx_b], rows_b, sem_b).start()

    def rmw(rows_b, idx_b, dst_b, sem_b):
        pltpu.make_async_copy(hp_hbm.at[idx_b], rows_b, sem_b).wait()

        @pl.loop(0, G // 16)
        def _(jg):
            dv16 = dst_b[pl.ds(jg * 16, 16)]
            for l in range(16):
                d = dv16[l]
                e = jg * 16 + l
                rv = [rows_b[e, pl.ds(k * 16, 16)] for k in range(8)]
                tv = [table[d, pl.ds(k * 16, 16)] for k in range(8)]
                for k in range(8):
                    table[d, pl.ds(k * 16, 16)] = jnp.maximum(tv[k], rv[k])

    @pl.when(nch > 0)
    def _():
        load_unpack_start(0, *bufs[0])

    def chunk_pair(it, _):
        for p in range(2):
            ch = it * 2 + p

            @pl.when(ch < nch)
            def _():
                @pl.when(ch + 1 < nch)
                def _():
                    load_unpack_start(ch + 1, *bufs[1 - p])

                rmw(*bufs[p])
        return 0

    lax.fori_loop(0, (nch + 1) // 2, chunk_pair, 0)
    pltpu.sync_copy(table.at[pl.ds(0, R)], agg_hbm.at[pl.ds(wid * R, R)])


@jax.jit
def _segmax(lists, counts, hp):
    kern = pl.kernel(
        _segmax_body,
        out_type=jax.ShapeDtypeStruct((NPAD, D), jnp.float32),
        mesh=_MESH,
        scratch_types=[
            pltpu.VMEM((R + 1, D), jnp.float32),
            pltpu.VMEM((G, D), jnp.float32),
            pltpu.VMEM((G, D), jnp.float32),
            pltpu.VMEM((G,), jnp.int32),
            pltpu.VMEM((G,), jnp.int32),
            pltpu.VMEM((G,), jnp.int32),
            pltpu.VMEM((G,), jnp.int32),
            pltpu.VMEM((G,), jnp.int32),
            pltpu.VMEM((16,), jnp.int32),
            pltpu.SemaphoreType.DMA,
            pltpu.SemaphoreType.DMA,
        ],
        compiler_params=_SC_PARAMS,
    )
    return kern(lists, counts, hp)


def _mm(a, w):
    return lax.dot_general(a, w, (((1,), (1,)), ((), ())),
                           preferred_element_type=jnp.float32,
                           precision=lax.Precision.HIGHEST)


def _dense1_body(x_ref, wp_ref, bp_ref, hp_ref):
    hp_ref[...] = jnp.maximum(_mm(x_ref[...], wp_ref[...]) + bp_ref[...], 0.0)


def _dense2_body(x_ref, a1_ref, ws_ref, wn_ref, b_ref, wp2_ref, bp2_ref,
                 h1_ref, hp2_ref):
    h = _mm(x_ref[...], ws_ref[...]) + _mm(a1_ref[...], wn_ref[...]) + b_ref[...]
    h = jnp.where(h >= 0.0, h, 0.01 * h)
    h1_ref[...] = h
    hp2_ref[...] = jnp.maximum(_mm(h, wp2_ref[...]) + bp2_ref[...], 0.0)


def _dense3_body(h1_ref, a2_ref, ws_ref, wn_ref, b_ref, o_ref):
    o_ref[...] = (_mm(h1_ref[...], ws_ref[...]) + _mm(a2_ref[...], wn_ref[...])
                  + b_ref[...])


@jax.jit
def _dense1(x, wp, bp):
    return pl.pallas_call(
        _dense1_body,
        out_shape=jax.ShapeDtypeStruct((N, D), jnp.float32),
    )(x, wp, bp)


@jax.jit
def _dense2(x, a1, ws, wn, b, wp2, bp2):
    return pl.pallas_call(
        _dense2_body,
        out_shape=(jax.ShapeDtypeStruct((N, D), jnp.float32),
                   jax.ShapeDtypeStruct((N, D), jnp.float32)),
    )(x, a1, ws, wn, b, wp2, bp2)


@jax.jit
def _dense3(h1, a2, ws, wn, b):
    return pl.pallas_call(
        _dense3_body,
        out_shape=jax.ShapeDtypeStruct((N, DOUT), jnp.float32),
    )(h1, a2, ws, wn, b)


def kernel(x, edge_index, W_pool1, b_pool1, W_self1, W_neigh1, b1,
           W_pool2, b_pool2, W_self2, W_neigh2, b2):
    src = edge_index[0]
    dst = edge_index[1]
    lists, counts = _scan(src, dst)
    hp1 = _dense1(x, W_pool1, b_pool1.reshape(1, D))
    agg1 = _segmax(lists, counts, hp1)[:N]
    h1, hp2 = _dense2(x, agg1, W_self1, W_neigh1, b1.reshape(1, D),
                      W_pool2, b_pool2.reshape(1, D))
    agg2 = _segmax(lists, counts, hp2)[:N]
    return _dense3(h1, agg2, W_self2, W_neigh2, b2.reshape(1, DOUT))


# speedup vs baseline: 5.2554x; 1.1054x over previous
import dataclasses
import functools

import jax
import jax.numpy as jnp
from jax import lax
from jax.experimental import pallas as pl
from jax.experimental.pallas import tpu as pltpu
from jax.experimental.pallas import tpu_sc as plsc

N = 10000
E = 320000
D = 128
DOUT = 16

NW = 32
R = 320
NPAD = NW * R
FLUSH = 1024
STAGE = FLUSH + 96
EPAD = (E // FLUSH + 2) * FLUSH
BLK = 16000
G = 128

_MESH = plsc.VectorSubcoreMesh(core_axis_name="c", subcore_axis_name="s")

_SC_PARAMS = pltpu.CompilerParams()
if "needs_layout_passes" in pltpu.CompilerParams.__dataclass_fields__:
    _SC_PARAMS = dataclasses.replace(_SC_PARAMS, needs_layout_passes=False)


def _wid():
    return lax.axis_index("s") * 2 + lax.axis_index("c")


DUMMY = 511


def _scan_body(src_hbm, dst_hbm, list_hbm, cnt_hbm,
               svb0, dvb0, svb1, dvb1, stage, cntv, sem0, sem1):
    wid = _wid()
    lo = wid * R
    nblk = E // BLK
    bufs = ((svb0, dvb0, sem0), (svb1, dvb1, sem1))

    def start_load(b, sv_b, dv_b, sem_b):
        pltpu.make_async_copy(src_hbm.at[pl.ds(b * BLK, BLK)], sv_b, sem_b).start()
        pltpu.make_async_copy(dst_hbm.at[pl.ds(b * BLK, BLK)], dv_b, sem_b).start()

    def wait_load(b, sv_b, dv_b, sem_b):
        pltpu.make_async_copy(src_hbm.at[pl.ds(b * BLK, BLK)], sv_b, sem_b).wait()
        pltpu.make_async_copy(dst_hbm.at[pl.ds(b * BLK, BLK)], dv_b, sem_b).wait()

    start_load(0, *bufs[0])

    def process_block(b, sv_b, dv_b, carry):
        def group4(jj, carry):
            f, nf = carry
            masks, packeds, pcs = [], [], []
            for u in range(4):
                j = jj * 4 + u
                dv = dv_b[pl.ds(j * 16, 16)]
                sv = sv_b[pl.ds(j * 16, 16)]
                dl = dv - lo
                m = dl.astype(jnp.uint32) < jnp.uint32(R)
                masks.append(m)
                packeds.append((sv << 9) | dl)
                pcs.append(plsc.all_reduce_population_count(m))
            for u in range(4):
                plsc.store_compressed(stage.at[pl.ds(f, 16)], packeds[u],
                                      mask=masks[u])
                f = f + pcs[u][0]
            do = f >= FLUSH

            @pl.when(do)
            def _():
                pltpu.sync_copy(stage.at[pl.ds(0, FLUSH)],
                                list_hbm.at[wid, pl.ds(nf * FLUSH, FLUSH)])
                for t in range(5):
                    stage[pl.ds(t * 16, 16)] = stage[pl.ds(FLUSH + t * 16, 16)]

            f = jnp.where(do, f - FLUSH, f)
            nf = nf + do.astype(jnp.int32)
            return f, nf

        return lax.fori_loop(0, BLK // 64, group4, carry)

    def block_pair(bb, carry):
        for p in range(2):
            b = bb * 2 + p

            @pl.when(b < nblk)
            def _():
                wait_load(b, *bufs[p])

                @pl.when(b + 1 < nblk)
                def _():
                    start_load(b + 1, *bufs[1 - p])

            carry = lax.cond(b < nblk,
                             lambda c: process_block(b, bufs[p][0], bufs[p][1], c),
                             lambda c: c, carry)
        return carry

    f, nf = lax.fori_loop(0, (nblk + 1) // 2, block_pair,
                          (jnp.int32(0), jnp.int32(0)))

    iota16 = lax.iota(jnp.int32, 16)

    @pl.loop(0, STAGE // 16)
    def _(j):
        g = j * 16
        sl = pl.ds(g, 16)
        stage[sl] = jnp.where(g + iota16 >= f, DUMMY, stage[sl])

    pltpu.sync_copy(stage.at[pl.ds(0, FLUSH)],
                    list_hbm.at[wid, pl.ds(nf * FLUSH, FLUSH)])
    cntv[...] = jnp.full((16,), nf * FLUSH + f, jnp.int32)
    pltpu.sync_copy(cntv, cnt_hbm.at[wid])


@jax.jit
def _scan(src, dst):
    kern = pl.kernel(
        _scan_body,
        out_type=(jax.ShapeDtypeStruct((NW, EPAD), jnp.int32),
                  jax.ShapeDtypeStruct((NW, 16), jnp.int32)),
        mesh=_MESH,
        scratch_types=[
            pltpu.VMEM((BLK,), jnp.int32),
            pltpu.VMEM((BLK,), jnp.int32),
            pltpu.VMEM((BLK,), jnp.int32),
            pltpu.VMEM((BLK,), jnp.int32),
            pltpu.VMEM((STAGE,), jnp.int32),
            pltpu.VMEM((16,), jnp.int32),
            pltpu.SemaphoreType.DMA,
            pltpu.SemaphoreType.DMA,
        ],
        compiler_params=_SC_PARAMS,
    )
    return kern(src, dst)


NBUF = 4


def _segmax_body(list_hbm, cnt_hbm, hp_hbm, agg_hbm,
                 table, rows, idxs, dsts, lch, cntv, sems):
    wid = _wid()
    zeros16 = jnp.zeros((16,), jnp.float32)

    @pl.loop(0, R + 1)
    def _(r):
        for k in range(8):
            table[r, pl.ds(k * 16, 16)] = zeros16

    pltpu.sync_copy(cnt_hbm.at[wid], cntv)
    cnt = jnp.max(cntv[...])
    nch = (cnt + G - 1) // G
    bufs = tuple((rows[p], idxs[p], dsts[p], sems[p]) for p in range(NBUF))

    def load_unpack_start(ch, rows_b, idx_b, dst_b, sem_b):
        pltpu.sync_copy(list_hbm.at[wid, pl.ds(ch * G, G)], lch)
        for j in range(G // 16):
            sl = pl.ds(j * 16, 16)
            v = lch[sl]
            idx_b[sl] = jnp.minimum(lax.shift_right_logical(v, 9), N - 1)
            dst_b[sl] = jnp.minimum(v & 511, R)
        pltpu.make_async_copy(hp_hbm.at[idx_b], rows_b, sem_b).start()

    def rmw(rows_b, idx_b, dst_b, sem_b):
        pltpu.make_async_copy(hp_hbm.at[idx_b], rows_b, sem_b).wait()

        @pl.loop(0, G // 16)
        def _(jg):
            dv16 = dst_b[pl.ds(jg * 16, 16)]
            for l in range(16):
                d = dv16[l]
                e = jg * 16 + l
                rv = [rows_b[e, pl.ds(k * 16, 16)] for k in range(8)]
                tv = [table[d, pl.ds(k * 16, 16)] for k in range(8)]
                for k in range(8):
                    table[d, pl.ds(k * 16, 16)] = jnp.maximum(tv[k], rv[k])

    for q in range(NBUF - 1):
        @pl.when(q < nch)
        def _(q=q):
            load_unpack_start(q, *bufs[q])

    def chunk_quad(it, _):
        for p in range(NBUF):
            ch = it * NBUF + p

            @pl.when(ch < nch)
            def _(p=p, ch=ch):
                @pl.when(ch + NBUF - 1 < nch)
                def _():
                    load_unpack_start(ch + NBUF - 1,
                                      *bufs[(p + NBUF - 1) % NBUF])

                rmw(*bufs[p])
        return 0

    lax.fori_loop(0, (nch + NBUF - 1) // NBUF, chunk_quad, 0)
    pltpu.sync_copy(table.at[pl.ds(0, R)], agg_hbm.at[pl.ds(wid * R, R)])


@jax.jit
def _segmax(lists, counts, hp):
    kern = pl.kernel(
        _segmax_body,
        out_type=jax.ShapeDtypeStruct((NPAD, D), jnp.float32),
        mesh=_MESH,
        scratch_types=[
            pltpu.VMEM((R + 1, D), jnp.float32),
            [pltpu.VMEM((G, D), jnp.float32) for _ in range(NBUF)],
            [pltpu.VMEM((G,), jnp.int32) for _ in range(NBUF)],
            [pltpu.VMEM((G,), jnp.int32) for _ in range(NBUF)],
            pltpu.VMEM((G,), jnp.int32),
            pltpu.VMEM((16,), jnp.int32),
            [pltpu.SemaphoreType.DMA for _ in range(NBUF)],
        ],
        compiler_params=_SC_PARAMS,
    )
    return kern(lists, counts, hp)


def _mm(a, w):
    return lax.dot_general(a, w, (((1,), (1,)), ((), ())),
                           preferred_element_type=jnp.float32,
                           precision=lax.Precision.HIGHEST)


def _dense1_body(x_ref, wp_ref, bp_ref, hp_ref):
    hp_ref[...] = jnp.maximum(_mm(x_ref[...], wp_ref[...]) + bp_ref[...], 0.0)


def _dense2_body(x_ref, a1_ref, ws_ref, wn_ref, b_ref, wp2_ref, bp2_ref,
                 h1_ref, hp2_ref):
    h = _mm(x_ref[...], ws_ref[...]) + _mm(a1_ref[...], wn_ref[...]) + b_ref[...]
    h = jnp.where(h >= 0.0, h, 0.01 * h)
    h1_ref[...] = h
    hp2_ref[...] = jnp.maximum(_mm(h, wp2_ref[...]) + bp2_ref[...], 0.0)


def _dense3_body(h1_ref, a2_ref, ws_ref, wn_ref, b_ref, o_ref):
    o_ref[...] = (_mm(h1_ref[...], ws_ref[...]) + _mm(a2_ref[...], wn_ref[...])
                  + b_ref[...])


@jax.jit
def _dense1(x, wp, bp):
    return pl.pallas_call(
        _dense1_body,
        out_shape=jax.ShapeDtypeStruct((N, D), jnp.float32),
    )(x, wp, bp)


@jax.jit
def _dense2(x, a1, ws, wn, b, wp2, bp2):
    return pl.pallas_call(
        _dense2_body,
        out_shape=(jax.ShapeDtypeStruct((N, D), jnp.float32),
                   jax.ShapeDtypeStruct((N, D), jnp.float32)),
    )(x, a1, ws, wn, b, wp2, bp2)


@jax.jit
def _dense3(h1, a2, ws, wn, b):
    return pl.pallas_call(
        _dense3_body,
        out_shape=jax.ShapeDtypeStruct((N, DOUT), jnp.float32),
    )(h1, a2, ws, wn, b)


def kernel(x, edge_index, W_pool1, b_pool1, W_self1, W_neigh1, b1,
           W_pool2, b_pool2, W_self2, W_neigh2, b2):
    src = edge_index[0]
    dst = edge_index[1]
    lists, counts = _scan(src, dst)
    hp1 = _dense1(x, W_pool1, b_pool1.reshape(1, D))
    agg1 = _segmax(lists, counts, hp1)[:N]
    h1, hp2 = _dense2(x, agg1, W_self1, W_neigh1, b1.reshape(1, D),
                      W_pool2, b_pool2.reshape(1, D))
    agg2 = _segmax(lists, counts, hp2)[:N]
    return _dense3(h1, agg2, W_self2, W_neigh2, b2.reshape(1, DOUT))


# speedup vs baseline: 5.7325x; 1.0908x over previous
import dataclasses
import functools

import jax
import jax.numpy as jnp
from jax import lax
from jax.experimental import pallas as pl
from jax.experimental.pallas import tpu as pltpu
from jax.experimental.pallas import tpu_sc as plsc

N = 10000
E = 320000
D = 128
DOUT = 16

NW = 32
R = 320
NPAD = NW * R
FLUSH = 1024
STAGE = FLUSH + 96
EPAD = (E // FLUSH + 2) * FLUSH
BLK = 16000
G = 128

_MESH = plsc.VectorSubcoreMesh(core_axis_name="c", subcore_axis_name="s")

_SC_PARAMS = pltpu.CompilerParams()
if "needs_layout_passes" in pltpu.CompilerParams.__dataclass_fields__:
    _SC_PARAMS = dataclasses.replace(_SC_PARAMS, needs_layout_passes=False)


def _wid():
    return lax.axis_index("s") * 2 + lax.axis_index("c")


DUMMY = 511


def _scan_body(src_hbm, dst_hbm, list_hbm, cnt_hbm,
               svb0, dvb0, svb1, dvb1, stage, cntv, sem0, sem1):
    wid = _wid()
    lo = wid * R
    nblk = E // BLK
    bufs = ((svb0, dvb0, sem0), (svb1, dvb1, sem1))

    def start_load(b, sv_b, dv_b, sem_b):
        pltpu.make_async_copy(src_hbm.at[pl.ds(b * BLK, BLK)], sv_b, sem_b).start()
        pltpu.make_async_copy(dst_hbm.at[pl.ds(b * BLK, BLK)], dv_b, sem_b).start()

    def wait_load(b, sv_b, dv_b, sem_b):
        pltpu.make_async_copy(src_hbm.at[pl.ds(b * BLK, BLK)], sv_b, sem_b).wait()
        pltpu.make_async_copy(dst_hbm.at[pl.ds(b * BLK, BLK)], dv_b, sem_b).wait()

    start_load(0, *bufs[0])

    def process_block(b, sv_b, dv_b, carry):
        def group4(jj, carry):
            f, nf = carry
            masks, packeds, pcs = [], [], []
            for u in range(4):
                j = jj * 4 + u
                dv = dv_b[pl.ds(j * 16, 16)]
                sv = sv_b[pl.ds(j * 16, 16)]
                dl = dv - lo
                m = dl.astype(jnp.uint32) < jnp.uint32(R)
                masks.append(m)
                packeds.append((sv << 9) | dl)
                pcs.append(plsc.all_reduce_population_count(m))
            for u in range(4):
                plsc.store_compressed(stage.at[pl.ds(f, 16)], packeds[u],
                                      mask=masks[u])
                f = f + pcs[u][0]
            do = f >= FLUSH

            @pl.when(do)
            def _():
                pltpu.sync_copy(stage.at[pl.ds(0, FLUSH)],
                                list_hbm.at[wid, pl.ds(nf * FLUSH, FLUSH)])
                for t in range(5):
                    stage[pl.ds(t * 16, 16)] = stage[pl.ds(FLUSH + t * 16, 16)]

            f = jnp.where(do, f - FLUSH, f)
            nf = nf + do.astype(jnp.int32)
            return f, nf

        return lax.fori_loop(0, BLK // 64, group4, carry)

    def block_pair(bb, carry):
        for p in range(2):
            b = bb * 2 + p

            @pl.when(b < nblk)
            def _():
                wait_load(b, *bufs[p])

                @pl.when(b + 1 < nblk)
                def _():
                    start_load(b + 1, *bufs[1 - p])

            carry = lax.cond(b < nblk,
                             lambda c: process_block(b, bufs[p][0], bufs[p][1], c),
                             lambda c: c, carry)
        return carry

    f, nf = lax.fori_loop(0, (nblk + 1) // 2, block_pair,
                          (jnp.int32(0), jnp.int32(0)))

    iota16 = lax.iota(jnp.int32, 16)

    @pl.loop(0, STAGE // 16)
    def _(j):
        g = j * 16
        sl = pl.ds(g, 16)
        stage[sl] = jnp.where(g + iota16 >= f, DUMMY, stage[sl])

    pltpu.sync_copy(stage.at[pl.ds(0, FLUSH)],
                    list_hbm.at[wid, pl.ds(nf * FLUSH, FLUSH)])
    cntv[...] = jnp.full((16,), nf * FLUSH + f, jnp.int32)
    pltpu.sync_copy(cntv, cnt_hbm.at[wid])


@jax.jit
def _scan(src, dst):
    kern = pl.kernel(
        _scan_body,
        out_type=(jax.ShapeDtypeStruct((NW, EPAD), jnp.int32),
                  jax.ShapeDtypeStruct((NW, 16), jnp.int32)),
        mesh=_MESH,
        scratch_types=[
            pltpu.VMEM((BLK,), jnp.int32),
            pltpu.VMEM((BLK,), jnp.int32),
            pltpu.VMEM((BLK,), jnp.int32),
            pltpu.VMEM((BLK,), jnp.int32),
            pltpu.VMEM((STAGE,), jnp.int32),
            pltpu.VMEM((16,), jnp.int32),
            pltpu.SemaphoreType.DMA,
            pltpu.SemaphoreType.DMA,
        ],
        compiler_params=_SC_PARAMS,
    )
    return kern(src, dst)


NBUF = 4


def _segmax_body(list_hbm, cnt_hbm, hp_hbm, agg_hbm,
                 table, rows, idxs, dsts, lchs, cntv, sems, lsems):
    wid = _wid()
    zeros16 = jnp.zeros((16,), jnp.float32)

    @pl.loop(0, R + 1)
    def _(r):
        for k in range(8):
            table[r, pl.ds(k * 16, 16)] = zeros16

    pltpu.sync_copy(cnt_hbm.at[wid], cntv)
    cnt = jnp.max(cntv[...])
    nch = (cnt + G - 1) // G

    def start_list(ch, p):
        pltpu.make_async_copy(list_hbm.at[wid, pl.ds(ch * G, G)],
                              lchs[p], lsems[p]).start()

    def unpack_gather(ch, p):
        pltpu.make_async_copy(list_hbm.at[wid, pl.ds(ch * G, G)],
                              lchs[p], lsems[p]).wait()
        for j in range(G // 16):
            sl = pl.ds(j * 16, 16)
            v = lchs[p][sl]
            idxs[p][sl] = jnp.minimum(lax.shift_right_logical(v, 9), N - 1)
            dsts[p][sl] = jnp.minimum(v & 511, R)
        pltpu.make_async_copy(hp_hbm.at[idxs[p]], rows[p], sems[p]).start()

    def rmw(p):
        pltpu.make_async_copy(hp_hbm.at[idxs[p]], rows[p], sems[p]).wait()
        rows_b, dst_b = rows[p], dsts[p]

        @pl.loop(0, G // 16)
        def _(jg):
            dv16 = dst_b[pl.ds(jg * 16, 16)]
            for l in range(16):
                d = dv16[l]
                e = jg * 16 + l
                rv = [rows_b[e, pl.ds(k * 16, 16)] for k in range(8)]
                tv = [table[d, pl.ds(k * 16, 16)] for k in range(8)]
                for k in range(8):
                    table[d, pl.ds(k * 16, 16)] = jnp.maximum(tv[k], rv[k])

    for q in range(3):
        @pl.when(q < nch)
        def _(q=q):
            start_list(q, q)

    for q in range(2):
        @pl.when(q < nch)
        def _(q=q):
            unpack_gather(q, q)

    def chunk_quad(it, _):
        for p in range(NBUF):
            ch = it * NBUF + p

            @pl.when(ch < nch)
            def _(p=p, ch=ch):
                @pl.when(ch + 3 < nch)
                def _():
                    start_list(ch + 3, (p + 3) % NBUF)

                @pl.when(ch + 2 < nch)
                def _():
                    unpack_gather(ch + 2, (p + 2) % NBUF)

                rmw(p)
        return 0

    lax.fori_loop(0, (nch + NBUF - 1) // NBUF, chunk_quad, 0)
    pltpu.sync_copy(table.at[pl.ds(0, R)], agg_hbm.at[pl.ds(wid * R, R)])


@jax.jit
def _segmax(lists, counts, hp):
    kern = pl.kernel(
        _segmax_body,
        out_type=jax.ShapeDtypeStruct((NPAD, D), jnp.float32),
        mesh=_MESH,
        scratch_types=[
            pltpu.VMEM((R + 1, D), jnp.float32),
            [pltpu.VMEM((G, D), jnp.float32) for _ in range(NBUF)],
            [pltpu.VMEM((G,), jnp.int32) for _ in range(NBUF)],
            [pltpu.VMEM((G,), jnp.int32) for _ in range(NBUF)],
            [pltpu.VMEM((G,), jnp.int32) for _ in range(NBUF)],
            pltpu.VMEM((16,), jnp.int32),
            [pltpu.SemaphoreType.DMA for _ in range(NBUF)],
            [pltpu.SemaphoreType.DMA for _ in range(NBUF)],
        ],
        compiler_params=_SC_PARAMS,
    )
    return kern(lists, counts, hp)


def _mm(a, w):
    return lax.dot_general(a, w, (((1,), (1,)), ((), ())),
                           preferred_element_type=jnp.float32,
                           precision=lax.Precision.HIGHEST)


def _dense1_body(x_ref, wp_ref, bp_ref, hp_ref):
    hp_ref[...] = jnp.maximum(_mm(x_ref[...], wp_ref[...]) + bp_ref[...], 0.0)


def _dense2_body(x_ref, a1_ref, ws_ref, wn_ref, b_ref, wp2_ref, bp2_ref,
                 h1_ref, hp2_ref):
    h = _mm(x_ref[...], ws_ref[...]) + _mm(a1_ref[...], wn_ref[...]) + b_ref[...]
    h = jnp.where(h >= 0.0, h, 0.01 * h)
    h1_ref[...] = h
    hp2_ref[...] = jnp.maximum(_mm(h, wp2_ref[...]) + bp2_ref[...], 0.0)


def _dense3_body(h1_ref, a2_ref, ws_ref, wn_ref, b_ref, o_ref):
    o_ref[...] = (_mm(h1_ref[...], ws_ref[...]) + _mm(a2_ref[...], wn_ref[...])
                  + b_ref[...])


@jax.jit
def _dense1(x, wp, bp):
    return pl.pallas_call(
        _dense1_body,
        out_shape=jax.ShapeDtypeStruct((N, D), jnp.float32),
    )(x, wp, bp)


@jax.jit
def _dense2(x, a1, ws, wn, b, wp2, bp2):
    return pl.pallas_call(
        _dense2_body,
        out_shape=(jax.ShapeDtypeStruct((N, D), jnp.float32),
                   jax.ShapeDtypeStruct((N, D), jnp.float32)),
    )(x, a1, ws, wn, b, wp2, bp2)


@jax.jit
def _dense3(h1, a2, ws, wn, b):
    return pl.pallas_call(
        _dense3_body,
        out_shape=jax.ShapeDtypeStruct((N, DOUT), jnp.float32),
    )(h1, a2, ws, wn, b)


def kernel(x, edge_index, W_pool1, b_pool1, W_self1, W_neigh1, b1,
           W_pool2, b_pool2, W_self2, W_neigh2, b2):
    src = edge_index[0]
    dst = edge_index[1]
    lists, counts = _scan(src, dst)
    hp1 = _dense1(x, W_pool1, b_pool1.reshape(1, D))
    agg1 = _segmax(lists, counts, hp1)[:N]
    h1, hp2 = _dense2(x, agg1, W_self1, W_neigh1, b1.reshape(1, D),
                      W_pool2, b_pool2.reshape(1, D))
    agg2 = _segmax(lists, counts, hp2)[:N]
    return _dense3(h1, agg2, W_self2, W_neigh2, b2.reshape(1, DOUT))


# speedup vs baseline: 6.2396x; 1.0885x over previous
import dataclasses
import functools

import jax
import jax.numpy as jnp
from jax import lax
from jax.experimental import pallas as pl
from jax.experimental.pallas import tpu as pltpu
from jax.experimental.pallas import tpu_sc as plsc

N = 10000
E = 320000
D = 128
DOUT = 16

NW = 32
R = 320
NPAD = NW * R
FLUSH = 1024
STAGE = FLUSH + 160
EPAD = (E // FLUSH + 2) * FLUSH
BLK = 16000
G = 128

_MESH = plsc.VectorSubcoreMesh(core_axis_name="c", subcore_axis_name="s")

_SC_PARAMS = pltpu.CompilerParams()
if "needs_layout_passes" in pltpu.CompilerParams.__dataclass_fields__:
    _SC_PARAMS = dataclasses.replace(_SC_PARAMS, needs_layout_passes=False)


def _wid():
    return lax.axis_index("s") * 2 + lax.axis_index("c")


DUMMY = 511


def _scan_body(src_hbm, dst_hbm, list_hbm, cnt_hbm,
               svb0, dvb0, svb1, dvb1, stage, cntv, sem0, sem1):
    wid = _wid()
    lo = wid * R
    nblk = E // BLK
    bufs = ((svb0, dvb0, sem0), (svb1, dvb1, sem1))

    def start_load(b, sv_b, dv_b, sem_b):
        pltpu.make_async_copy(src_hbm.at[pl.ds(b * BLK, BLK)], sv_b, sem_b).start()
        pltpu.make_async_copy(dst_hbm.at[pl.ds(b * BLK, BLK)], dv_b, sem_b).start()

    def wait_load(b, sv_b, dv_b, sem_b):
        pltpu.make_async_copy(src_hbm.at[pl.ds(b * BLK, BLK)], sv_b, sem_b).wait()
        pltpu.make_async_copy(dst_hbm.at[pl.ds(b * BLK, BLK)], dv_b, sem_b).wait()

    start_load(0, *bufs[0])

    def process_block(b, sv_b, dv_b, carry):
        def group8(jj, carry):
            f, nf = carry
            masks, packeds, pcs = [], [], []
            for u in range(8):
                j = jj * 8 + u
                dv = dv_b[pl.ds(j * 16, 16)]
                sv = sv_b[pl.ds(j * 16, 16)]
                dl = dv - lo
                m = dl.astype(jnp.uint32) < jnp.uint32(R)
                masks.append(m)
                packeds.append((sv << 9) | dl)
                pcs.append(plsc.all_reduce_population_count(m))
            for u in range(8):
                plsc.store_compressed(stage.at[pl.ds(f, 16)], packeds[u],
                                      mask=masks[u])
                f = f + pcs[u][0]
            do = f >= FLUSH

            @pl.when(do)
            def _():
                pltpu.sync_copy(stage.at[pl.ds(0, FLUSH)],
                                list_hbm.at[wid, pl.ds(nf * FLUSH, FLUSH)])
                for t in range(9):
                    stage[pl.ds(t * 16, 16)] = stage[pl.ds(FLUSH + t * 16, 16)]

            f = jnp.where(do, f - FLUSH, f)
            nf = nf + do.astype(jnp.int32)
            return f, nf

        return lax.fori_loop(0, BLK // 128, group8, carry)

    def block_pair(bb, carry):
        for p in range(2):
            b = bb * 2 + p

            @pl.when(b < nblk)
            def _():
                wait_load(b, *bufs[p])

                @pl.when(b + 1 < nblk)
                def _():
                    start_load(b + 1, *bufs[1 - p])

            carry = lax.cond(b < nblk,
                             lambda c: process_block(b, bufs[p][0], bufs[p][1], c),
                             lambda c: c, carry)
        return carry

    f, nf = lax.fori_loop(0, (nblk + 1) // 2, block_pair,
                          (jnp.int32(0), jnp.int32(0)))

    iota16 = lax.iota(jnp.int32, 16)

    @pl.loop(0, STAGE // 16)
    def _(j):
        g = j * 16
        sl = pl.ds(g, 16)
        stage[sl] = jnp.where(g + iota16 >= f, DUMMY, stage[sl])

    pltpu.sync_copy(stage.at[pl.ds(0, FLUSH)],
                    list_hbm.at[wid, pl.ds(nf * FLUSH, FLUSH)])
    cntv[...] = jnp.full((16,), nf * FLUSH + f, jnp.int32)
    pltpu.sync_copy(cntv, cnt_hbm.at[wid])


@jax.jit
def _scan(src, dst):
    kern = pl.kernel(
        _scan_body,
        out_type=(jax.ShapeDtypeStruct((NW, EPAD), jnp.int32),
                  jax.ShapeDtypeStruct((NW, 16), jnp.int32)),
        mesh=_MESH,
        scratch_types=[
            pltpu.VMEM((BLK,), jnp.int32),
            pltpu.VMEM((BLK,), jnp.int32),
            pltpu.VMEM((BLK,), jnp.int32),
            pltpu.VMEM((BLK,), jnp.int32),
            pltpu.VMEM((STAGE,), jnp.int32),
            pltpu.VMEM((16,), jnp.int32),
            pltpu.SemaphoreType.DMA,
            pltpu.SemaphoreType.DMA,
        ],
        compiler_params=_SC_PARAMS,
    )
    return kern(src, dst)


NBUF = 4


def _segmax_body(list_hbm, cnt_hbm, hp_hbm, agg_hbm,
                 table, rows, idxs, dsts, lchs, cntv, sems, lsems):
    wid = _wid()
    zeros16 = jnp.zeros((16,), jnp.float32)

    @pl.loop(0, R + 1)
    def _(r):
        for k in range(8):
            table[r, pl.ds(k * 16, 16)] = zeros16

    pltpu.sync_copy(cnt_hbm.at[wid], cntv)
    cnt = jnp.max(cntv[...])
    nch = (cnt + G - 1) // G

    def start_list(ch, p):
        pltpu.make_async_copy(list_hbm.at[wid, pl.ds(ch * G, G)],
                              lchs[p], lsems[p]).start()

    def unpack_gather(ch, p):
        pltpu.make_async_copy(list_hbm.at[wid, pl.ds(ch * G, G)],
                              lchs[p], lsems[p]).wait()
        for j in range(G // 16):
            sl = pl.ds(j * 16, 16)
            v = lchs[p][sl]
            idxs[p][sl] = jnp.minimum(lax.shift_right_logical(v, 9), N - 1)
            dsts[p][sl] = jnp.minimum(v & 511, R)
        pltpu.make_async_copy(hp_hbm.at[idxs[p]], rows[p], sems[p]).start()

    def rmw(p):
        pltpu.make_async_copy(hp_hbm.at[idxs[p]], rows[p], sems[p]).wait()
        rows_b, dst_b = rows[p], dsts[p]

        @pl.loop(0, G // 16)
        def _(jg):
            dv16 = dst_b[pl.ds(jg * 16, 16)]
            for l in range(16):
                d = dv16[l]
                e = jg * 16 + l
                rv = [rows_b[e, pl.ds(k * 16, 16)] for k in range(8)]
                tv = [table[d, pl.ds(k * 16, 16)] for k in range(8)]
                for k in range(8):
                    table[d, pl.ds(k * 16, 16)] = jnp.maximum(tv[k], rv[k])

    for q in range(3):
        @pl.when(q < nch)
        def _(q=q):
            start_list(q, q)

    for q in range(2):
        @pl.when(q < nch)
        def _(q=q):
            unpack_gather(q, q)

    def chunk_quad(it, _):
        for p in range(NBUF):
            ch = it * NBUF + p

            @pl.when(ch < nch)
            def _(p=p, ch=ch):
                @pl.when(ch + 3 < nch)
                def _():
                    start_list(ch + 3, (p + 3) % NBUF)

                @pl.when(ch + 2 < nch)
                def _():
                    unpack_gather(ch + 2, (p + 2) % NBUF)

                rmw(p)
        return 0

    lax.fori_loop(0, (nch + NBUF - 1) // NBUF, chunk_quad, 0)
    pltpu.sync_copy(table.at[pl.ds(0, R)], agg_hbm.at[pl.ds(wid * R, R)])


@jax.jit
def _segmax(lists, counts, hp):
    kern = pl.kernel(
        _segmax_body,
        out_type=jax.ShapeDtypeStruct((NPAD, D), jnp.float32),
        mesh=_MESH,
        scratch_types=[
            pltpu.VMEM((R + 1, D), jnp.float32),
            [pltpu.VMEM((G, D), jnp.float32) for _ in range(NBUF)],
            [pltpu.VMEM((G,), jnp.int32) for _ in range(NBUF)],
            [pltpu.VMEM((G,), jnp.int32) for _ in range(NBUF)],
            [pltpu.VMEM((G,), jnp.int32) for _ in range(NBUF)],
            pltpu.VMEM((16,), jnp.int32),
            [pltpu.SemaphoreType.DMA for _ in range(NBUF)],
            [pltpu.SemaphoreType.DMA for _ in range(NBUF)],
        ],
        compiler_params=_SC_PARAMS,
    )
    return kern(lists, counts, hp)


def _mm(a, w):
    return lax.dot_general(a, w, (((1,), (1,)), ((), ())),
                           preferred_element_type=jnp.float32,
                           precision=lax.Precision.HIGHEST)


def _dense1_body(x_ref, wp_ref, bp_ref, hp_ref):
    hp_ref[...] = jnp.maximum(_mm(x_ref[...], wp_ref[...]) + bp_ref[...], 0.0)


def _dense2_body(x_ref, a1_ref, ws_ref, wn_ref, b_ref, wp2_ref, bp2_ref,
                 h1_ref, hp2_ref):
    h = _mm(x_ref[...], ws_ref[...]) + _mm(a1_ref[...], wn_ref[...]) + b_ref[...]
    h = jnp.where(h >= 0.0, h, 0.01 * h)
    h1_ref[...] = h
    hp2_ref[...] = jnp.maximum(_mm(h, wp2_ref[...]) + bp2_ref[...], 0.0)


def _dense3_body(h1_ref, a2_ref, ws_ref, wn_ref, b_ref, o_ref):
    o_ref[...] = (_mm(h1_ref[...], ws_ref[...]) + _mm(a2_ref[...], wn_ref[...])
                  + b_ref[...])


@jax.jit
def _dense1(x, wp, bp):
    return pl.pallas_call(
        _dense1_body,
        out_shape=jax.ShapeDtypeStruct((N, D), jnp.float32),
    )(x, wp, bp)


@jax.jit
def _dense2(x, a1, ws, wn, b, wp2, bp2):
    return pl.pallas_call(
        _dense2_body,
        out_shape=(jax.ShapeDtypeStruct((N, D), jnp.float32),
                   jax.ShapeDtypeStruct((N, D), jnp.float32)),
    )(x, a1, ws, wn, b, wp2, bp2)


@jax.jit
def _dense3(h1, a2, ws, wn, b):
    return pl.pallas_call(
        _dense3_body,
        out_shape=jax.ShapeDtypeStruct((N, DOUT), jnp.float32),
    )(h1, a2, ws, wn, b)


def kernel(x, edge_index, W_pool1, b_pool1, W_self1, W_neigh1, b1,
           W_pool2, b_pool2, W_self2, W_neigh2, b2):
    src = edge_index[0]
    dst = edge_index[1]
    lists, counts = _scan(src, dst)
    hp1 = _dense1(x, W_pool1, b_pool1.reshape(1, D))
    agg1 = _segmax(lists, counts, hp1)[:N]
    h1, hp2 = _dense2(x, agg1, W_self1, W_neigh1, b1.reshape(1, D),
                      W_pool2, b_pool2.reshape(1, D))
    agg2 = _segmax(lists, counts, hp2)[:N]
    return _dense3(h1, agg2, W_self2, W_neigh2, b2.reshape(1, DOUT))
